# baseline (speedup 1.0000x reference)
import functools

import jax
import jax.numpy as jnp
from jax import lax
from jax.experimental import pallas as pl
from jax.experimental.pallas import tpu as pltpu
from jax.experimental.pallas import tpu_sc as plsc

VOCAB = 1000000
DIM = 64
BATCH = 16384

NC = 2
NS = 16
LANES = 16
NW = NC * NS
B_PER_W = BATCH // NW
CHUNK = 128
N_CHUNKS = B_PER_W // CHUNK
GROUPS = B_PER_W // LANES


def _sc_logits_kernel(center_hbm, context_hbm, win_hbm, wout_hbm, out_hbm,
                      cidx_v, oidx_v, a_v, b_v, tr_v, logit_v, sem_a, sem_b):
    wid = lax.axis_index("s") * NC + lax.axis_index("c")
    base = wid * B_PER_W

    pltpu.sync_copy(center_hbm.at[pl.ds(base, B_PER_W)], cidx_v)
    pltpu.sync_copy(context_hbm.at[pl.ds(base, B_PER_W)], oidx_v)

    copies = []
    for j in range(N_CHUNKS):
        sl = pl.ds(j * CHUNK, CHUNK)
        copies.append(
            pltpu.async_copy(win_hbm.at[cidx_v.at[sl]], a_v.at[sl], sem_a))
        copies.append(
            pltpu.async_copy(wout_hbm.at[oidx_v.at[sl]], b_v.at[sl], sem_b))
    for cp in copies:
        cp.wait()

    lane = lax.iota(jnp.int32, LANES)
    lane16 = lane * LANES

    def group_body(g, _):
        row0 = g * LANES
        for r in range(LANES):
            row = row0 + r
            s = (a_v[row, pl.ds(0, LANES)] * b_v[row, pl.ds(0, LANES)]
                 + a_v[row, pl.ds(LANES, LANES)] * b_v[row, pl.ds(LANES, LANES)]
                 + a_v[row, pl.ds(2 * LANES, LANES)] * b_v[row, pl.ds(2 * LANES, LANES)]
                 + a_v[row, pl.ds(3 * LANES, LANES)] * b_v[row, pl.ds(3 * LANES, LANES)])
            plsc.store_scatter(tr_v, [lane16 + r], s)
        acc = tr_v[pl.ds(0, LANES)]
        for j in range(1, LANES):
            acc = acc + tr_v[pl.ds(j * LANES, LANES)]
        logit_v[pl.ds(row0, LANES)] = acc
        return 0

    lax.fori_loop(0, GROUPS, group_body, 0)

    pltpu.sync_copy(logit_v, out_hbm.at[pl.ds(base, B_PER_W)])


_sc_logits = functools.partial(
    pl.kernel,
    mesh=plsc.VectorSubcoreMesh(core_axis_name="c", subcore_axis_name="s"),
    out_type=jax.ShapeDtypeStruct((BATCH,), jnp.float32),
    compiler_params=pltpu.CompilerParams(
        needs_layout_passes=False, use_tc_tiling_on_sc=False),
    scratch_types=[
        pltpu.VMEM((B_PER_W,), jnp.int32),
        pltpu.VMEM((B_PER_W,), jnp.int32),
        pltpu.VMEM((B_PER_W, DIM), jnp.float32),
        pltpu.VMEM((B_PER_W, DIM), jnp.float32),
        pltpu.VMEM((LANES * LANES,), jnp.float32),
        pltpu.VMEM((B_PER_W,), jnp.float32),
        pltpu.SemaphoreType.DMA,
        pltpu.SemaphoreType.DMA,
    ],
)(_sc_logits_kernel)


def _bce_kernel(logits_ref, labels_ref, out_ref):
    x = logits_ref[...]
    y = labels_ref[...]
    per = jnp.maximum(x, 0.0) - x * y + jnp.log1p(jnp.exp(-jnp.abs(x)))
    out_ref[0, 0] = jnp.sum(per) / BATCH


def kernel(center_words, context_words, labels, W_in, W_out):
    logits = _sc_logits(center_words.astype(jnp.int32),
                        context_words.astype(jnp.int32), W_in, W_out)
    loss = pl.pallas_call(
        _bce_kernel,
        out_shape=jax.ShapeDtypeStruct((1, 1), jnp.float32),
        in_specs=[
            pl.BlockSpec(memory_space=pltpu.VMEM),
            pl.BlockSpec(memory_space=pltpu.VMEM),
        ],
        out_specs=pl.BlockSpec(memory_space=pltpu.SMEM),
    )(logits.reshape(128, 128), labels.reshape(128, 128))
    return loss[0, 0]
